# Initial kernel scaffold; baseline (speedup 1.0000x reference)
import jax
import jax.numpy as jnp
from jax.experimental import pallas as pl


def kernel(query, ref):
    raise NotImplementedError("write your pallas kernel here")


# speedup vs baseline: 11.2167x; 11.2167x over previous
import jax
import jax.numpy as jnp
from jax.experimental import pallas as pl
from jax.experimental.pallas import tpu as pltpu


def _chamfer_body(q_ref, rt_ref, out_ref, min_ref, *, nq, nr, inv_qk):
    qi = pl.program_id(0)
    ri = pl.program_id(1)
    q = q_ref[...]
    rt = rt_ref[...]
    dots = jnp.dot(q, rt, preferred_element_type=jnp.float32)
    r2 = jnp.sum(rt * rt, axis=0)
    part = r2[None, :] - 2.0 * dots
    m = jnp.min(part, axis=1, keepdims=True)

    @pl.when(ri == 0)
    def _init():
        min_ref[...] = m

    @pl.when(ri != 0)
    def _acc():
        min_ref[...] = jnp.minimum(min_ref[...], m)

    @pl.when(ri == nr - 1)
    def _final():
        q2 = jnp.sum(q * q, axis=1, keepdims=True)
        partial = jnp.sum(min_ref[...] + q2, axis=(0, 1), keepdims=True) * inv_qk

        @pl.when(qi == 0)
        def _first():
            out_ref[...] = partial

        @pl.when(qi != 0)
        def _rest():
            out_ref[...] += partial


def kernel(query, ref):
    q_n, d = query.shape
    r_n, _ = ref.shape
    bq = min(512, q_n)
    br = min(2048, r_n)
    nq, nr = q_n // bq, r_n // br

    import functools

    body = functools.partial(
        _chamfer_body, nq=nq, nr=nr, inv_qk=1.0 / float(q_n)
    )
    out = pl.pallas_call(
        body,
        grid=(nq, nr),
        in_specs=[
            pl.BlockSpec((bq, d), lambda qi, ri: (qi, 0)),
            pl.BlockSpec((d, br), lambda qi, ri: (0, ri)),
        ],
        out_specs=pl.BlockSpec((1, 1), lambda qi, ri: (0, 0)),
        out_shape=jax.ShapeDtypeStruct((1, 1), jnp.float32),
        scratch_shapes=[pltpu.VMEM((bq, 1), jnp.float32)],
    )(query, ref.T)
    return out[0, 0]
